# baseline (speedup 1.0000x reference)
import jax
import jax.numpy as jnp
from jax.experimental import pallas as pl

V = 1000
N = 3
JB = 1


def _fill_kernel(alpha_ref, batch_ref, out_ref, tgt_ref):
    a0 = alpha_ref[0, 0]
    a1 = alpha_ref[0, 1]
    a2 = alpha_ref[0, 2]
    s_full = (a0 + a1 + a2) * (1.0 / V)
    s_last = (a0 + a1) * (1.0 / V)

    i = pl.program_id(0)
    out_ref[...] = jnp.zeros(out_ref.shape, jnp.float32) + s_full

    @pl.when(i == pl.num_programs(0) - 1)
    def _():
        out_ref[JB - 1:, :, :] = (
            jnp.zeros((1,) + out_ref.shape[1:], jnp.float32) + s_last)

    @pl.when(i == 0)
    def _():
        tgt_ref[...] = jnp.transpose(batch_ref[:, N - 1:])


def kernel(batch, TEXT, alpha):
    B, bptt = batch.shape
    n_preds = bptt - (N - 1) + 1
    n_tgt = n_preds - 1

    out_t, tgt_t = pl.pallas_call(
        _fill_kernel,
        grid=(n_preds // JB,),
        in_specs=[
            pl.BlockSpec((1, 3), lambda i: (0, 0)),
            pl.BlockSpec((B, bptt), lambda i: (0, 0)),
        ],
        out_specs=[
            pl.BlockSpec((JB, V, B), lambda i: (i, 0, 0)),
            pl.BlockSpec((n_tgt, B), lambda i: (0, 0)),
        ],
        out_shape=[
            jax.ShapeDtypeStruct((n_preds, V, B), jnp.float32),
            jax.ShapeDtypeStruct((n_tgt, B), batch.dtype),
        ],
    )(alpha.reshape(1, 3), batch)
    outputs = jnp.transpose(out_t, (2, 0, 1))
    targets = jnp.transpose(tgt_t)
    return outputs, targets


# speedup vs baseline: 1.0064x; 1.0064x over previous
import jax
import jax.numpy as jnp
from jax.experimental import pallas as pl
from jax.experimental.pallas import tpu as pltpu

V = 1000
N = 3


def _fill_kernel(alpha_ref, batch_hbm, out_ref, tgt_ref, batch_vmem, sem):
    a0 = alpha_ref[0, 0]
    a1 = alpha_ref[0, 1]
    a2 = alpha_ref[0, 2]
    s_full = (a0 + a1 + a2) * (1.0 / V)
    s_last = (a0 + a1) * (1.0 / V)

    i = pl.program_id(0)
    last = pl.num_programs(0) - 1
    val = jnp.where(i == last, s_last, s_full)
    out_ref[...] = jnp.zeros(out_ref.shape, jnp.float32) + val

    @pl.when(i == 0)
    def _():
        pltpu.make_async_copy(batch_hbm, batch_vmem, sem).start()

    @pl.when(i == last)
    def _():
        pltpu.make_async_copy(batch_hbm, batch_vmem, sem).wait()
        tgt_ref[...] = jnp.transpose(batch_vmem[:, N - 1:])


def kernel(batch, TEXT, alpha):
    B, bptt = batch.shape
    n_preds = bptt - (N - 1) + 1
    n_tgt = n_preds - 1

    out_t, tgt_t = pl.pallas_call(
        _fill_kernel,
        grid=(n_preds,),
        in_specs=[
            pl.BlockSpec((1, 3), lambda i: (0, 0)),
            pl.BlockSpec(memory_space=pltpu.MemorySpace.HBM),
        ],
        out_specs=[
            pl.BlockSpec((1, V, B), lambda i: (i, 0, 0)),
            pl.BlockSpec((n_tgt, B), lambda i: (0, 0)),
        ],
        out_shape=[
            jax.ShapeDtypeStruct((n_preds, V, B), jnp.float32),
            jax.ShapeDtypeStruct((n_tgt, B), batch.dtype),
        ],
        scratch_shapes=[
            pltpu.VMEM((B, bptt), batch.dtype),
            pltpu.SemaphoreType.DMA,
        ],
    )(alpha.reshape(1, 3), batch)
    outputs = jnp.transpose(out_t, (2, 0, 1))
    targets = jnp.transpose(tgt_t)
    return outputs, targets
